# baseline (speedup 1.0000x reference)
import functools

import jax
import jax.numpy as jnp
from jax import lax
from jax.experimental import pallas as pl
from jax.experimental.pallas import tpu as pltpu
from jax.experimental.pallas import tpu_sc as plsc

NC = 2
NS = 16
NW = NC * NS
D = 128

f32 = jnp.float32
i32 = jnp.int32


def _pick_br(n, cands=(512, 400, 256, 200, 128, 80, 40, 8)):
    for c in cands:
        if n % c == 0:
            return c
    return 8


def _proj_kernel(x_ref, wac_ref, bac_ref, wb_ref, bb_ref, ac_ref, b_ref):
    xb = x_ref[...]
    ac_ref[...] = jnp.dot(xb, wac_ref[...], preferred_element_type=f32) + bac_ref[...]
    b_ref[...] = jnp.dot(xb, wb_ref[...], preferred_element_type=f32) + bb_ref[...]


def _node_projections(nf, W, b):
    V = nf.shape[0]
    BR = _pick_br(V)
    wac = jnp.concatenate([W[0], W[4]], axis=1)
    bac = jnp.concatenate([b[0], b[4]])[None, :]
    wb = W[1]
    bb = b[1][None, :]
    return pl.pallas_call(
        _proj_kernel,
        grid=(V // BR,),
        in_specs=[
            pl.BlockSpec((BR, D), lambda i: (i, 0)),
            pl.BlockSpec((D, 2 * D), lambda i: (0, 0)),
            pl.BlockSpec((1, 2 * D), lambda i: (0, 0)),
            pl.BlockSpec((D, D), lambda i: (0, 0)),
            pl.BlockSpec((1, D), lambda i: (0, 0)),
        ],
        out_specs=[
            pl.BlockSpec((BR, 2 * D), lambda i: (i, 0)),
            pl.BlockSpec((BR, D), lambda i: (i, 0)),
        ],
        out_shape=[
            jax.ShapeDtypeStruct((V, 2 * D), f32),
            jax.ShapeDtypeStruct((V, D), f32),
        ],
    )(nf, wac, bac, wb, bb)


def _ln(v, g, bta):
    mu = jnp.mean(v, axis=-1, keepdims=True)
    var = jnp.mean((v - mu) ** 2, axis=-1, keepdims=True)
    return (v - mu) / jnp.sqrt(var + 1e-5) * g + bta


def _edge_kernel(gac_ref, gb_ref, ef_ref, w2_ref, b2_ref, g_ref, bta_ref,
                 efn_ref, ps_ref):
    gac = gac_ref[...]
    ef = ef_ref[...]
    ew = jnp.dot(ef, w2_ref[...], preferred_element_type=f32) + b2_ref[...]
    m = gac[:, :D] + gb_ref[...] + ew
    sigma = jax.nn.sigmoid(m)
    ps_ref[...] = jnp.concatenate([gac[:, D:] * sigma, sigma], axis=1)
    efn_ref[...] = ef + jax.nn.silu(_ln(m, g_ref[...], bta_ref[...]))


def _edge_stage(gac, gb, ef, W2e, b2e, g, bta):
    Eg = ef.shape[0]
    BR = _pick_br(Eg)
    return pl.pallas_call(
        _edge_kernel,
        grid=(Eg // BR,),
        in_specs=[
            pl.BlockSpec((BR, 2 * D), lambda i: (i, 0)),
            pl.BlockSpec((BR, D), lambda i: (i, 0)),
            pl.BlockSpec((BR, D), lambda i: (i, 0)),
            pl.BlockSpec((D, D), lambda i: (0, 0)),
            pl.BlockSpec((1, D), lambda i: (0, 0)),
            pl.BlockSpec((1, D), lambda i: (0, 0)),
            pl.BlockSpec((1, D), lambda i: (0, 0)),
        ],
        out_specs=[
            pl.BlockSpec((BR, D), lambda i: (i, 0)),
            pl.BlockSpec((BR, 2 * D), lambda i: (i, 0)),
        ],
        out_shape=[
            jax.ShapeDtypeStruct((Eg, D), f32),
            jax.ShapeDtypeStruct((Eg, 2 * D), f32),
        ],
    )(gac, gb, ef, W2e, b2e[None, :], g[None, :], bta[None, :])


def _node_kernel(nd_ref, nf_ref, w3_ref, b3_ref, g_ref, bta_ref, out_ref):
    nf = nf_ref[...]
    nd = nd_ref[...]
    h = nd[:, :D] / (nd[:, D:] + 1e-6)
    xo = jnp.dot(nf, w3_ref[...], preferred_element_type=f32) + b3_ref[...] + h
    out_ref[...] = nf + jax.nn.silu(_ln(xo, g_ref[...], bta_ref[...]))


def _node_stage(nd, nf, W3, b3, g, bta):
    V = nf.shape[0]
    BR = _pick_br(V)
    return pl.pallas_call(
        _node_kernel,
        grid=(V // BR,),
        in_specs=[
            pl.BlockSpec((BR, 2 * D), lambda i: (i, 0)),
            pl.BlockSpec((BR, D), lambda i: (i, 0)),
            pl.BlockSpec((D, D), lambda i: (0, 0)),
            pl.BlockSpec((1, D), lambda i: (0, 0)),
            pl.BlockSpec((1, D), lambda i: (0, 0)),
            pl.BlockSpec((1, D), lambda i: (0, 0)),
        ],
        out_specs=pl.BlockSpec((BR, D), lambda i: (i, 0)),
        out_shape=jax.ShapeDtypeStruct((V, D), f32),
    )(nd, nf, W3, b3[None, :], g[None, :], bta[None, :])


def _gather_two(acmat, bmat, src, dst):
    Eg = src.shape[0]
    assert Eg % NW == 0
    rows_per_tile = Eg // NW
    nfull, tail = divmod(rows_per_tile, 128)
    mesh = plsc.VectorSubcoreMesh(core_axis_name="c", subcore_axis_name="s")

    npair, odd = divmod(nfull, 2)

    @functools.partial(
        pl.kernel,
        out_type=(
            jax.ShapeDtypeStruct((Eg, 2 * D), f32),
            jax.ShapeDtypeStruct((Eg, D), f32),
        ),
        mesh=mesh,
        scratch_types=[
            pltpu.VMEM((128,), i32),
            pltpu.VMEM((128,), i32),
            pltpu.VMEM((128,), i32),
            pltpu.VMEM((128,), i32),
            pltpu.VMEM((128, 2 * D), f32),
            pltpu.VMEM((128, D), f32),
            pltpu.VMEM((128, 2 * D), f32),
            pltpu.VMEM((128, D), f32),
            pltpu.VMEM((max(tail, 8),), i32),
            pltpu.VMEM((max(tail, 8),), i32),
            pltpu.VMEM((max(tail, 8), 2 * D), f32),
            pltpu.VMEM((max(tail, 8), D), f32),
            pltpu.SemaphoreType.DMA,
            pltpu.SemaphoreType.DMA,
            pltpu.SemaphoreType.DMA,
            pltpu.SemaphoreType.DMA,
        ],
    )
    def k(ac_hbm, b_hbm, src_hbm, dst_hbm, gac_hbm, gb_hbm,
          sidxA, didxA, sidxB, didxB, acbufA, bbufA, acbufB, bbufB,
          sidx_t, didx_t, acbuf_t, bbuf_t, semGA, semGB, semWA, semWB):
        wid = lax.axis_index("s") * NC + lax.axis_index("c")
        base = wid * rows_per_tile

        def gathers(cb, sidx, didx, acbuf, bbuf, semG):
            pltpu.sync_copy(src_hbm.at[pl.ds(cb, 128)], sidx)
            pltpu.sync_copy(dst_hbm.at[pl.ds(cb, 128)], didx)
            pltpu.async_copy(ac_hbm.at[sidx], acbuf, semG)
            pltpu.async_copy(b_hbm.at[didx], bbuf, semG)

        def wait_gathers(sidx, didx, acbuf, bbuf, semG):
            pltpu.make_async_copy(ac_hbm.at[sidx], acbuf, semG).wait()
            pltpu.make_async_copy(b_hbm.at[didx], bbuf, semG).wait()

        def writes(cb, acbuf, bbuf, semW):
            pltpu.async_copy(acbuf, gac_hbm.at[pl.ds(cb, 128)], semW)
            pltpu.async_copy(bbuf, gb_hbm.at[pl.ds(cb, 128)], semW)

        def wait_writes(cb, acbuf, bbuf, semW):
            pltpu.make_async_copy(acbuf, gac_hbm.at[pl.ds(cb, 128)],
                                  semW).wait()
            pltpu.make_async_copy(bbuf, gb_hbm.at[pl.ds(cb, 128)],
                                  semW).wait()

        def pair(t, _):
            cbA = base + 2 * t * 128
            cbB = cbA + 128

            @pl.when(t > 0)
            def _():
                wait_writes(cbA, acbufA, bbufA, semWA)
            gathers(cbA, sidxA, didxA, acbufA, bbufA, semGA)

            @pl.when(t > 0)
            def _():
                wait_writes(cbB, acbufB, bbufB, semWB)
            gathers(cbB, sidxB, didxB, acbufB, bbufB, semGB)

            wait_gathers(sidxA, didxA, acbufA, bbufA, semGA)
            writes(cbA, acbufA, bbufA, semWA)
            wait_gathers(sidxB, didxB, acbufB, bbufB, semGB)
            writes(cbB, acbufB, bbufB, semWB)
            return 0

        if npair:
            lax.fori_loop(0, npair, pair, 0)
            wait_writes(base, acbufA, bbufA, semWA)
            wait_writes(base, acbufB, bbufB, semWB)
        if odd:
            cb = base + (nfull - 1) * 128
            pltpu.sync_copy(src_hbm.at[pl.ds(cb, 128)], sidxA)
            pltpu.sync_copy(dst_hbm.at[pl.ds(cb, 128)], didxA)
            pltpu.sync_copy(ac_hbm.at[sidxA], acbufA)
            pltpu.sync_copy(b_hbm.at[didxA], bbufA)
            pltpu.sync_copy(acbufA, gac_hbm.at[pl.ds(cb, 128)])
            pltpu.sync_copy(bbufA, gb_hbm.at[pl.ds(cb, 128)])
        if tail:
            cb = base + nfull * 128
            pltpu.sync_copy(src_hbm.at[pl.ds(cb, tail)], sidx_t)
            pltpu.sync_copy(dst_hbm.at[pl.ds(cb, tail)], didx_t)
            pltpu.sync_copy(ac_hbm.at[sidx_t], acbuf_t)
            pltpu.sync_copy(b_hbm.at[didx_t], bbuf_t)
            pltpu.sync_copy(acbuf_t, gac_hbm.at[pl.ds(cb, tail)])
            pltpu.sync_copy(bbuf_t, gb_hbm.at[pl.ds(cb, tail)])

    return k(acmat, bmat, src, dst)


def _largest_div(n, cap):
    for c in range(cap - cap % 8, 0, -8):
        if n % c == 0:
            return c
    return 8


def _segment_sum(vals, dst, V):
    Eg = dst.shape[0]
    NIDS = 2000
    assert Eg % NIDS == 0
    rpt = -(-V // (NW * 8)) * 8
    Vp = rpt * NW
    SR = _largest_div(rpt, 256)
    nsub = rpt // SR
    LL = Eg + 128
    CAP = 11264
    Q = min(4, nsub)
    spc = -(-nsub // Q)
    qrows = SR * spc
    CAPQ = CAP + 16 * Q + 192
    BIG = jnp.int32(2**30)
    mesh = plsc.VectorSubcoreMesh(core_axis_name="c", subcore_axis_name="s")

    @functools.partial(
        pl.kernel,
        out_type=(
            jax.ShapeDtypeStruct((Vp, 2 * D), f32),
            jax.ShapeDtypeStruct((NW * LL,), i32),
            jax.ShapeDtypeStruct((NW * LL,), i32),
        ),
        mesh=mesh,
        scratch_types=[
            pltpu.VMEM((NIDS,), i32),
            pltpu.VMEM((3072,), i32),
            pltpu.VMEM((3072,), i32),
            pltpu.VMEM((128,), i32),
            pltpu.VMEM((128,), i32),
            pltpu.VMEM((128,), i32),
            pltpu.VMEM((128,), i32),
            pltpu.VMEM((128, 2 * D), f32),
            pltpu.VMEM((SR + 1, 2 * D), f32),
            pltpu.VMEM((CAP,), i32),
            pltpu.VMEM((CAP,), i32),
            pltpu.VMEM((CAPQ,), i32),
            pltpu.VMEM((NIDS,), i32),
            pltpu.SemaphoreType.DMA,
            pltpu.SemaphoreType.DMA,
            pltpu.SemaphoreType.DMA,
        ],
        compiler_params=pltpu.CompilerParams(needs_layout_passes=False),
    )
    def k(vals_hbm, dst_hbm, zero_hbm, out_hbm, il_hbm, dl_hbm,
          idsb, idxb, offb, listi, listd, idxs, offs, rowb, acc,
          listv_i, listv_d, qbuf, idsb2, semA, semB, semG):
        cid = lax.axis_index("c")
        sid = lax.axis_index("s")
        wid = sid * NC + cid
        lo = wid * rpt
        hi = lo + rpt
        lbase = wid * LL
        lane = lax.iota(i32, 16)

        def move_tail(nfl):
            for kk in range(8):
                vi = idxb[pl.ds(nfl * 128 + kk * 16, 16)]
                vo = offb[pl.ds(nfl * 128 + kk * 16, 16)]
                idxb[pl.ds(kk * 16, 16)] = vi
                offb[pl.ds(kk * 16, 16)] = vo

        NB = Eg // NIDS
        assert NB % 2 == 0

        def start_load(b, buf, sem):
            pltpu.async_copy(dst_hbm.at[pl.ds(b * NIDS, NIDS)], buf, sem)

        def wait_load(b, buf, sem):
            pltpu.make_async_copy(dst_hbm.at[pl.ds(b * NIDS, NIDS)], buf,
                                  sem).wait()

        def scan_block(b, carry, buf):
            cur, hcur = carry

            def batch(bstart, nb, c):
                inrs, gidxs, idss, curs = [], [], [], [c]
                for q in range(nb):
                    g = bstart + q
                    ids = buf[pl.ds(g * 16, 16)]
                    inr = (ids >= lo) & (ids < hi)
                    inrs.append(inr)
                    idss.append(ids)
                    gidxs.append(b * NIDS + g * 16 + lane)
                    curs.append(curs[-1] + plsc.all_reduce_population_count(inr)[0])
                for q in range(nb):
                    plsc.store_compressed(idxb.at[pl.ds(curs[q], 16)],
                                          gidxs[q], mask=inrs[q])
                    plsc.store_compressed(offb.at[pl.ds(curs[q], 16)],
                                          idss[q], mask=inrs[q])
                return curs[-1]

            def grp8(t, c):
                return batch(t * 8, 8, c)

            ng8, ngr = divmod(NIDS // 16, 8)
            cur = lax.fori_loop(0, ng8, grp8, cur)
            if ngr:
                cur = batch(ng8 * 8, ngr, cur)
            nfl = cur >> 7

            def fl(j, _):
                b2 = hcur + j * 128

                @pl.when(b2 + 128 <= CAP)
                def _():
                    for kk in range(8):
                        listv_i[pl.ds(b2 + kk * 16, 16)] = (
                            idxb[pl.ds(j * 128 + kk * 16, 16)])
                        listv_d[pl.ds(b2 + kk * 16, 16)] = (
                            offb[pl.ds(j * 128 + kk * 16, 16)])

                @pl.when(b2 + 128 > CAP)
                def _():
                    o = pl.multiple_of(lbase + b2, 128)
                    pltpu.sync_copy(idxb.at[pl.ds(j * 128, 128)],
                                    il_hbm.at[pl.ds(o, 128)])
                    pltpu.sync_copy(offb.at[pl.ds(j * 128, 128)],
                                    dl_hbm.at[pl.ds(o, 128)])
                return 0

            lax.fori_loop(0, nfl, fl, 0)

            @pl.when(nfl > 0)
            def _():
                move_tail(nfl)

            return cur & 127, hcur + nfl * 128

        start_load(0, idsb, semA)

        def scan_pair(t, carry):
            b0 = 2 * t
            start_load(b0 + 1, idsb2, semB)
            wait_load(b0, idsb, semA)
            carry = scan_block(b0, carry, idsb)
            start_load((b0 + 2) % NB, idsb, semA)
            wait_load(b0 + 1, idsb2, semB)
            carry = scan_block(b0 + 1, carry, idsb2)
            return carry

        cur, hcur = lax.fori_loop(0, NB // 2, scan_pair,
                                  (jnp.int32(0), jnp.int32(0)))
        wait_load(0, idsb, semA)

        a0 = (cur >> 4) << 4
        rem = cur - a0
        gi = idxb[pl.ds(a0, 16)]
        go = offb[pl.ds(a0, 16)]
        idxb[pl.ds(a0, 16)] = jnp.where(lane < rem, gi, 0)
        offb[pl.ds(a0, 16)] = jnp.where(lane < rem, go, BIG)
        for kk in range(1, 8):
            idxb[pl.ds(a0 + 16 * kk, 16)] = jnp.zeros((16,), i32)
            offb[pl.ds(a0 + 16 * kk, 16)] = jnp.zeros((16,), i32) + BIG

        @pl.when((cur > 0) & (hcur + 128 <= CAP))
        def _():
            for kk in range(8):
                listv_i[pl.ds(hcur + kk * 16, 16)] = idxb[pl.ds(kk * 16, 16)]
                listv_d[pl.ds(hcur + kk * 16, 16)] = offb[pl.ds(kk * 16, 16)]

        @pl.when((cur > 0) & (hcur + 128 > CAP))
        def _():
            o = pl.multiple_of(lbase + hcur, 128)
            pltpu.sync_copy(idxb.at[pl.ds(0, 128)], il_hbm.at[pl.ds(o, 128)])
            pltpu.sync_copy(offb.at[pl.ds(0, 128)], dl_hbm.at[pl.ds(o, 128)])

        nlb = (hcur + jnp.where(cur > 0, 128, 0)) >> 7

        cached_ok = nlb * 128 <= CAP
        ngl = nlb * 8

        def count_grp(g, cnts):
            off = listv_d[pl.ds(g * 16, 16)] - lo
            out = []
            for q in range(Q):
                mq = (off >= q * qrows) & (off < (q + 1) * qrows)
                out.append(cnts[q] + plsc.all_reduce_population_count(mq)[0])
            return tuple(out)

        zero_cnts = tuple(jnp.int32(0) for _ in range(Q))
        cnts = lax.cond(
            cached_ok,
            lambda: lax.fori_loop(0, ngl, count_grp, zero_cnts),
            lambda: zero_cnts)
        qst = []
        run = jnp.int32(0)
        for q in range(Q):
            qst.append(run + 16 * q)
            run = run + cnts[q]

        def dist_grp(g, curs):
            d = listv_d[pl.ds(g * 16, 16)]
            iv = listv_i[pl.ds(g * 16, 16)]
            off = d - lo
            v = lax.shift_left(iv, 13) | (off & 8191)
            out = []
            for q in range(Q):
                mq = (off >= q * qrows) & (off < (q + 1) * qrows)
                plsc.store_compressed(qbuf.at[pl.ds(curs[q], 16)], v, mask=mq)
                out.append(curs[q] + plsc.all_reduce_population_count(mq)[0])
            return tuple(out)

        def do_dist():
            lax.fori_loop(0, ngl, dist_grp, tuple(qst))
            return jnp.int32(0)

        lax.cond(cached_ok, do_dist, lambda: jnp.int32(0))

        def stage_and_start():
            for kk in range(8):
                idxs[pl.ds(kk * 16, 16)] = idxb[pl.ds(kk * 16, 16)]
                offs[pl.ds(kk * 16, 16)] = offb[pl.ds(kk * 16, 16)]
            pltpu.async_copy(vals_hbm.at[idxs], rowb, semG)

        def finish_pending():
            pltpu.make_async_copy(vals_hbm.at[idxs], rowb, semG).wait()

            def item_grp(gq, _):
                offv = offs[pl.ds(gq * 16, 16)]
                for L in range(16):
                    off = offv[L]
                    r = gq * 16 + L
                    for kk in range(16):
                        plsc.addupdate(acc.at[off, pl.ds(kk * 16, 16)],
                                       rowb[r, pl.ds(kk * 16, 16)])
                return 0

            lax.fori_loop(0, 8, item_grp, 0)

        def subchunk(s, _):
            slo = lo + s * SR
            shi = slo + SR
            pltpu.sync_copy(zero_hbm, acc)
            qs = s // spc
            qst_s = qst[0]
            qln_s = cnts[0]
            for q in range(1, Q):
                qst_s = jnp.where(qs == q, qst[q], qst_s)
                qln_s = jnp.where(qs == q, cnts[q], qln_s)
            sbase = s * SR

            def qblock(bb, carry):
                cur2, pending = carry
                ms, iis, dis, curs = [], [], [], [cur2]
                for g in range(8):
                    ei = bb * 128 + g * 16
                    v = qbuf[pl.ds(qst_s + ei, 16)]
                    offv = v & 8191
                    iv = lax.shift_right_logical(v, 13)
                    m = ((ei + lane < qln_s) & (offv >= sbase)
                         & (offv < sbase + SR))
                    ms.append(m)
                    iis.append(iv)
                    dis.append(offv - sbase)
                    curs.append(curs[-1] + plsc.all_reduce_population_count(m)[0])
                for g in range(8):
                    plsc.store_compressed(idxb.at[pl.ds(curs[g], 16)],
                                          iis[g], mask=ms[g])
                    plsc.store_compressed(offb.at[pl.ds(curs[g], 16)],
                                          dis[g], mask=ms[g])
                cur2 = curs[-1]
                nfl = cur2 >> 7

                @pl.when(nfl > 0)
                def _():
                    @pl.when(pending > 0)
                    def _():
                        finish_pending()

                    stage_and_start()

                    def fl2(j, __):
                        finish_pending()
                        move_tail(j)
                        stage_and_start()
                        return 0

                    lax.fori_loop(1, nfl, fl2, 0)
                    move_tail(nfl)

                return cur2 & 127, jnp.where(nfl > 0, 1, pending)

            def list_block(blk, carry):
                cur2, pending = carry
                base = blk * 128
                use_cache = base + 128 <= CAP

                @pl.when(jnp.logical_not(use_cache))
                def _():
                    o = pl.multiple_of(lbase + base, 128)
                    pltpu.sync_copy(il_hbm.at[pl.ds(o, 128)], listi)
                    pltpu.sync_copy(dl_hbm.at[pl.ds(o, 128)], listd)

                cb = jnp.minimum(base, CAP - 128)
                ms, iis, dis, curs = [], [], [], [cur2]
                for g in range(8):
                    di = jnp.where(use_cache,
                                   listv_d[pl.ds(cb + g * 16, 16)],
                                   listd[pl.ds(g * 16, 16)])
                    ii = jnp.where(use_cache,
                                   listv_i[pl.ds(cb + g * 16, 16)],
                                   listi[pl.ds(g * 16, 16)])
                    m = (di >= slo) & (di < shi)
                    ms.append(m)
                    iis.append(ii)
                    dis.append(di - slo)
                    curs.append(curs[-1] + plsc.all_reduce_population_count(m)[0])
                for g in range(8):
                    plsc.store_compressed(idxb.at[pl.ds(curs[g], 16)],
                                          iis[g], mask=ms[g])
                    plsc.store_compressed(offb.at[pl.ds(curs[g], 16)],
                                          dis[g], mask=ms[g])
                cur2 = curs[-1]
                nfl = cur2 >> 7

                @pl.when(nfl > 0)
                def _():
                    @pl.when(pending > 0)
                    def _():
                        finish_pending()

                    stage_and_start()

                    def fl2(j, __):
                        finish_pending()
                        move_tail(j)
                        stage_and_start()
                        return 0

                    lax.fori_loop(1, nfl, fl2, 0)
                    move_tail(nfl)

                return cur2 & 127, jnp.where(nfl > 0, 1, pending)

            nqb = (qln_s + 127) >> 7
            cur2, pending = lax.cond(
                cached_ok,
                lambda: lax.fori_loop(0, nqb, qblock,
                                      (jnp.int32(0), jnp.int32(0))),
                lambda: lax.fori_loop(0, nlb, list_block,
                                      (jnp.int32(0), jnp.int32(0))))

            @pl.when(pending > 0)
            def _():
                finish_pending()

            a0 = (cur2 >> 4) << 4
            rem2 = cur2 - a0
            gi2 = idxb[pl.ds(a0, 16)]
            go2 = offb[pl.ds(a0, 16)]
            idxb[pl.ds(a0, 16)] = jnp.where(lane < rem2, gi2, 0)
            offb[pl.ds(a0, 16)] = jnp.where(lane < rem2, go2, SR)
            for kk in range(1, 8):
                idxb[pl.ds(a0 + 16 * kk, 16)] = jnp.zeros((16,), i32)
                offb[pl.ds(a0 + 16 * kk, 16)] = jnp.zeros((16,), i32) + SR

            @pl.when(cur2 > 0)
            def _():
                stage_and_start()
                finish_pending()

            pltpu.sync_copy(acc.at[pl.ds(0, SR)],
                            out_hbm.at[pl.ds(pl.multiple_of(slo, 8), SR)])
            return 0

        lax.fori_loop(0, nsub, subchunk, 0)

    zero = jnp.zeros((SR + 1, 2 * D), f32)
    out, _, _ = k(vals, dst, zero)
    return out


def _egc_layer(nf, ef, src, dst, W, b, lng, lnb):
    acmat, bmat = _node_projections(nf, W, b)
    gac, gb = _gather_two(acmat, bmat, src, dst)
    ef_new, ps = _edge_stage(gac, gb, ef, W[2], b[2], lng[1], lnb[1])
    nd = _segment_sum(ps, dst, nf.shape[0])[: nf.shape[0]]
    nf_new = _node_stage(nd, nf, W[3], b[3], lng[0], lnb[0])
    return nf_new, ef_new


def kernel(x, y, z, edge_index, lg_edge_index, W1, b1, ln_g1, ln_b1,
           W2, b2, ln_g2, ln_b2):
    src, dst = edge_index[0], edge_index[1]
    ls, ld = lg_edge_index[0], lg_edge_index[1]
    x_new, m = _egc_layer(x, y, src, dst, W1, b1, ln_g1, ln_b1)
    y_new, z_new = _egc_layer(m, z, ls, ld, W2, b2, ln_g2, ln_b2)
    return (x_new, y_new, z_new)
